# baseline (speedup 1.0000x reference)
import functools

import jax
import jax.numpy as jnp
from jax import lax
from jax.experimental import pallas as pl
from jax.experimental.pallas import tpu as pltpu
from jax.experimental.pallas import tpu_sc as plsc

N = 10000
NP4 = 2500
NPAD = 10240
E = 320000
EPAD = 327680
NC, NS = 2, 16
NW = NC * NS
NPT = NPAD // NS
TB = 512
NT = EPAD // (NW * TB)
NBUF = 4


def _sc_deg_c0_body(dstp, srcp, ones_h, zer_h, out_deg, out_c0,
                    dst_v, src_v, ones_v, upd_v, deg_sh, c0_sh):
    c = lax.axis_index("c")
    s = lax.axis_index("s")
    w = c * NS + s
    pltpu.sync_copy(zer_h, deg_sh.at[pl.ds(s * NPT, NPT)])
    pltpu.sync_copy(zer_h, c0_sh.at[pl.ds(s * NPT, NPT)])
    pltpu.sync_copy(ones_h, ones_v)
    pltpu.sync_copy(dstp.at[pl.ds(w * NT * TB, NT * TB)], dst_v)
    pltpu.sync_copy(srcp.at[pl.ds(w * NT * TB, NT * TB)], src_v)
    plsc.subcore_barrier()

    def body(j, carry):
        pltpu.sync_copy(ones_v, deg_sh.at[dst_v.at[pl.ds(j * TB, TB)]],
                        add=True)
        cnt = jnp.int32(0)
        for k in range(TB // 16):
            d16 = dst_v[pl.ds(j * TB + k * 16, 16)]
            hit = d16 == 0
            upd_v[pl.ds(k * 16, 16)] = jnp.where(
                hit, jnp.float32(1.0), jnp.float32(0.0))
            cnt = cnt + jnp.sum(
                jnp.where(hit, jnp.int32(1), jnp.int32(0)))

        @pl.when(cnt > 0)
        def _():
            pltpu.sync_copy(upd_v, c0_sh.at[src_v.at[pl.ds(j * TB, TB)]],
                            add=True)

        return carry

    lax.fori_loop(0, NT, body, 0)
    plsc.subcore_barrier()
    pltpu.sync_copy(deg_sh.at[pl.ds(s * NPT, NPT)],
                    out_deg.at[c, pl.ds(s * NPT, NPT)])
    pltpu.sync_copy(c0_sh.at[pl.ds(s * NPT, NPT)],
                    out_c0.at[c, pl.ds(s * NPT, NPT)])


def _sc_gather_scatter_body(srcp, dstp, hs, zer2_h, out_agg,
                            src_v, dst_v, rows, agg_sh, sems):
    c = lax.axis_index("c")
    s = lax.axis_index("s")
    w = c * NS + s
    pltpu.sync_copy(zer2_h, agg_sh.at[pl.ds(s * NPT, NPT)])
    pltpu.sync_copy(srcp.at[pl.ds(w * NT * TB, NT * TB)], src_v)
    pltpu.sync_copy(dstp.at[pl.ds(w * NT * TB, NT * TB)], dst_v)
    plsc.subcore_barrier()

    for b in range(NBUF):
        pltpu.async_copy(hs.at[src_v.at[pl.ds(b * TB, TB)]], rows[b], sems[b])

    def body(g, carry):
        for b in range(NBUF):
            j = NBUF * g + b
            pltpu.make_async_copy(
                hs.at[src_v.at[pl.ds(j * TB, TB)]], rows[b], sems[b]).wait()
            pltpu.sync_copy(rows[b], agg_sh.at[dst_v.at[pl.ds(j * TB, TB)]],
                            add=True)

            @pl.when(g < NT // NBUF - 1)
            def _():
                pltpu.async_copy(
                    hs.at[src_v.at[pl.ds((j + NBUF) * TB, TB)]], rows[b], sems[b])

        return carry

    lax.fori_loop(0, NT // NBUF, body, 0)
    plsc.subcore_barrier()
    pltpu.sync_copy(agg_sh.at[pl.ds(s * NPT, NPT)],
                    out_agg.at[c, pl.ds(s * NPT, NPT)])


@functools.cache
def _sc_kernels():
    mesh = plsc.VectorSubcoreMesh(core_axis_name="c", subcore_axis_name="s")
    params = pltpu.CompilerParams(
        use_tc_tiling_on_sc=False, needs_layout_passes=False,
        skip_device_barrier=True)
    deg_c0 = pl.kernel(
        _sc_deg_c0_body,
        mesh=mesh,
        out_type=[
            jax.ShapeDtypeStruct((NC, NPAD), jnp.float32),
            jax.ShapeDtypeStruct((NC, NPAD), jnp.float32),
        ],
        scratch_types=[
            pltpu.VMEM((NT * TB,), jnp.int32),
            pltpu.VMEM((NT * TB,), jnp.int32),
            pltpu.VMEM((TB,), jnp.float32),
            pltpu.VMEM((TB,), jnp.float32),
            pltpu.VMEM_SHARED((NPAD,), jnp.float32),
            pltpu.VMEM_SHARED((NPAD,), jnp.float32),
        ],
        compiler_params=params,
    )
    gather_scatter = pl.kernel(
        _sc_gather_scatter_body,
        mesh=mesh,
        out_type=[
            jax.ShapeDtypeStruct((NC, NPAD, 32), jnp.float32),
        ],
        scratch_types=[
            pltpu.VMEM((NT * TB,), jnp.int32),
            pltpu.VMEM((NT * TB,), jnp.int32),
            [pltpu.VMEM((TB, 32), jnp.float32) for _ in range(NBUF)],
            pltpu.VMEM_SHARED((NPAD, 32), jnp.float32),
            [pltpu.SemaphoreType.DMA for _ in range(NBUF)],
        ],
        compiler_params=params,
    )
    return deg_c0, gather_scatter


def _tc_h(x_ref, w1_ref, out_ref):
    out_ref[...] = jnp.dot(
        x_ref[...], w1_ref[...], preferred_element_type=jnp.float32)


def _tc_hs(h_ref, degp_ref, out_ref):
    dp = degp_ref[...]
    deg = dp[0] + dp[1] + jnp.float32(1.0)
    dinv = lax.rsqrt(deg)
    out_ref[...] = h_ref[...] * dinv[:N].reshape(N, 1)


def _tc_final(aggp_ref, hs_ref, degp_ref, c0p_ref, b1_ref, w2_ref, b2_ref,
              out_ref):
    dp = degp_ref[...]
    deg = dp[0] + dp[1] + jnp.float32(1.0)
    dinv = lax.rsqrt(deg)
    dv = dinv[:N].reshape(N, 1)
    ap = aggp_ref[...]
    agg = ap[0] + ap[1]
    r1 = jnp.maximum(
        (agg[:N] + hs_ref[...]) * dv + b1_ref[...], jnp.float32(0.0))
    cp = c0p_ref[...]
    c0 = (cp[0] + cp[1])[:N].reshape(N, 1)
    w0 = c0 * dv
    sacc = jnp.sum(r1 * w0, axis=0, keepdims=True)
    d0 = dinv[0:1].reshape(1, 1)
    z = sacc * d0 + r1[0:1] * (d0 * d0)
    out_ref[...] = (
        jnp.dot(z, w2_ref[...], preferred_element_type=jnp.float32)
        + b2_ref[...])


def kernel(x, edge_index, W1, b1, W2, b2):
    ei = edge_index.astype(jnp.int32)
    npe = EPAD - E
    srcp = jnp.concatenate(
        [ei[0], jnp.zeros((npe,), jnp.int32)])
    dstp = jnp.concatenate(
        [ei[1], jnp.full((npe,), N, jnp.int32)])
    ones_h = jnp.ones((TB,), jnp.float32)
    zer_h = jnp.zeros((NPT,), jnp.float32)
    zer2_h = jnp.zeros((NPT, 32), jnp.float32)

    deg_c0, gather_scatter = _sc_kernels()
    degp, c0p = deg_c0(dstp, srcp, ones_h, zer_h)

    h = pl.pallas_call(
        _tc_h,
        out_shape=jax.ShapeDtypeStruct((N, 32), jnp.float32),
    )(x, W1)
    hs = pl.pallas_call(
        _tc_hs,
        out_shape=jax.ShapeDtypeStruct((N, 32), jnp.float32),
    )(h, degp)

    (out_agg,) = gather_scatter(srcp, dstp, hs, zer2_h)

    out = pl.pallas_call(
        _tc_final,
        out_shape=jax.ShapeDtypeStruct((1, 64), jnp.float32),
    )(out_agg, hs, degp, c0p, b1.reshape(1, 32), W2, b2.reshape(1, 64))
    return out.reshape(64)


# speedup vs baseline: 1.8764x; 1.8764x over previous
import functools

import jax
import jax.numpy as jnp
from jax import lax
from jax.experimental import pallas as pl
from jax.experimental.pallas import tpu as pltpu
from jax.experimental.pallas import tpu_sc as plsc

N = 10000
NP4 = 2500
NPAD = 10240
E = 320000
NC, NS = 2, 16
NW = NC * NS
NPT = NPAD // NS
EPW = E // NW
TB = 400
NT = EPW // TB
NBUF = 5


def _sc_deg_c0_body(ei, ones_h, zer_h, out_deg, out_c0,
                    dst_v, src_v, ones_v, upd_v, deg_sh, c0_sh):
    c = lax.axis_index("c")
    s = lax.axis_index("s")
    w = c * NS + s
    pltpu.sync_copy(zer_h, deg_sh.at[pl.ds(s * NPT, NPT)])
    pltpu.sync_copy(zer_h, c0_sh.at[pl.ds(s * NPT, NPT)])
    pltpu.sync_copy(ones_h, ones_v)
    pltpu.sync_copy(ei.at[1, pl.ds(w * EPW, EPW)], dst_v)
    pltpu.sync_copy(ei.at[0, pl.ds(w * EPW, EPW)], src_v)
    plsc.subcore_barrier()

    def body(j, carry):
        pltpu.sync_copy(ones_v, deg_sh.at[dst_v.at[pl.ds(j * TB, TB)]],
                        add=True)
        cnt = jnp.int32(0)
        for k in range(TB // 16):
            d16 = dst_v[pl.ds(j * TB + k * 16, 16)]
            hit = d16 == 0
            upd_v[pl.ds(k * 16, 16)] = jnp.where(
                hit, jnp.float32(1.0), jnp.float32(0.0))
            cnt = cnt + jnp.sum(
                jnp.where(hit, jnp.int32(1), jnp.int32(0)))

        @pl.when(cnt > 0)
        def _():
            pltpu.sync_copy(upd_v, c0_sh.at[src_v.at[pl.ds(j * TB, TB)]],
                            add=True)

        return carry

    lax.fori_loop(0, NT, body, 0)
    plsc.subcore_barrier()
    pltpu.sync_copy(deg_sh.at[pl.ds(s * NPT, NPT)],
                    out_deg.at[c, pl.ds(s * NPT, NPT)])
    pltpu.sync_copy(c0_sh.at[pl.ds(s * NPT, NPT)],
                    out_c0.at[c, pl.ds(s * NPT, NPT)])


def _sc_gather_scatter_body(ei, hs, zer2_h, out_agg,
                            src_v, dst_v, rows, agg_sh, sems):
    c = lax.axis_index("c")
    s = lax.axis_index("s")
    w = c * NS + s
    pltpu.sync_copy(zer2_h, agg_sh.at[pl.ds(s * NPT, NPT)])
    pltpu.sync_copy(ei.at[0, pl.ds(w * EPW, EPW)], src_v)
    pltpu.sync_copy(ei.at[1, pl.ds(w * EPW, EPW)], dst_v)
    plsc.subcore_barrier()

    for b in range(NBUF):
        pltpu.async_copy(hs.at[src_v.at[pl.ds(b * TB, TB)]], rows[b], sems[b])

    def body(g, carry):
        for b in range(NBUF):
            j = NBUF * g + b
            pltpu.make_async_copy(
                hs.at[src_v.at[pl.ds(j * TB, TB)]], rows[b], sems[b]).wait()
            pltpu.sync_copy(rows[b], agg_sh.at[dst_v.at[pl.ds(j * TB, TB)]],
                            add=True)

            @pl.when(g < NT // NBUF - 1)
            def _():
                pltpu.async_copy(
                    hs.at[src_v.at[pl.ds((j + NBUF) * TB, TB)]], rows[b], sems[b])

        return carry

    lax.fori_loop(0, NT // NBUF, body, 0)
    plsc.subcore_barrier()
    pltpu.sync_copy(agg_sh.at[pl.ds(s * NPT, NPT)],
                    out_agg.at[c, pl.ds(s * NPT, NPT)])


@functools.cache
def _sc_kernels():
    mesh = plsc.VectorSubcoreMesh(core_axis_name="c", subcore_axis_name="s")
    params = pltpu.CompilerParams(
        use_tc_tiling_on_sc=False, needs_layout_passes=False,
        skip_device_barrier=True)
    deg_c0 = pl.kernel(
        _sc_deg_c0_body,
        mesh=mesh,
        out_type=[
            jax.ShapeDtypeStruct((NC, NPAD), jnp.float32),
            jax.ShapeDtypeStruct((NC, NPAD), jnp.float32),
        ],
        scratch_types=[
            pltpu.VMEM((EPW,), jnp.int32),
            pltpu.VMEM((EPW,), jnp.int32),
            pltpu.VMEM((TB,), jnp.float32),
            pltpu.VMEM((TB,), jnp.float32),
            pltpu.VMEM_SHARED((NPAD,), jnp.float32),
            pltpu.VMEM_SHARED((NPAD,), jnp.float32),
        ],
        compiler_params=params,
    )
    gather_scatter = pl.kernel(
        _sc_gather_scatter_body,
        mesh=mesh,
        out_type=[
            jax.ShapeDtypeStruct((NC, NPAD, 32), jnp.float32),
        ],
        scratch_types=[
            pltpu.VMEM((EPW,), jnp.int32),
            pltpu.VMEM((EPW,), jnp.int32),
            [pltpu.VMEM((TB, 32), jnp.float32) for _ in range(NBUF)],
            pltpu.VMEM_SHARED((NPAD, 32), jnp.float32),
            [pltpu.SemaphoreType.DMA for _ in range(NBUF)],
        ],
        compiler_params=params,
    )
    return deg_c0, gather_scatter


def _tc_h(x_ref, w1_ref, out_ref):
    out_ref[...] = jnp.dot(
        x_ref[...], w1_ref[...], preferred_element_type=jnp.float32)


def _tc_hs(h_ref, degp_ref, out_ref):
    dp = degp_ref[...]
    deg = dp[0] + dp[1] + jnp.float32(1.0)
    dinv = lax.rsqrt(deg)
    out_ref[...] = h_ref[...] * dinv[:N].reshape(N, 1)


def _tc_final(aggp_ref, hs_ref, degp_ref, c0p_ref, b1_ref, w2_ref, b2_ref,
              out_ref):
    dp = degp_ref[...]
    deg = dp[0] + dp[1] + jnp.float32(1.0)
    dinv = lax.rsqrt(deg)
    dv = dinv[:N].reshape(N, 1)
    ap = aggp_ref[...]
    agg = ap[0] + ap[1]
    r1 = jnp.maximum(
        (agg[:N] + hs_ref[...]) * dv + b1_ref[...], jnp.float32(0.0))
    cp = c0p_ref[...]
    c0 = (cp[0] + cp[1])[:N].reshape(N, 1)
    w0 = c0 * dv
    sacc = jnp.sum(r1 * w0, axis=0, keepdims=True)
    d0 = dinv[0:1].reshape(1, 1)
    z = sacc * d0 + r1[0:1] * (d0 * d0)
    out_ref[...] = (
        jnp.dot(z, w2_ref[...], preferred_element_type=jnp.float32)
        + b2_ref[...])


def kernel(x, edge_index, W1, b1, W2, b2):
    ei = edge_index.astype(jnp.int32)
    ones_h = jnp.ones((TB,), jnp.float32)
    zer_h = jnp.zeros((NPT,), jnp.float32)
    zer2_h = jnp.zeros((NPT, 32), jnp.float32)

    deg_c0, gather_scatter = _sc_kernels()
    degp, c0p = deg_c0(ei, ones_h, zer_h)

    h = pl.pallas_call(
        _tc_h,
        out_shape=jax.ShapeDtypeStruct((N, 32), jnp.float32),
    )(x, W1)
    hs = pl.pallas_call(
        _tc_hs,
        out_shape=jax.ShapeDtypeStruct((N, 32), jnp.float32),
    )(h, degp)

    (out_agg,) = gather_scatter(ei, hs, zer2_h)

    out = pl.pallas_call(
        _tc_final,
        out_shape=jax.ShapeDtypeStruct((1, 64), jnp.float32),
    )(out_agg, hs, degp, c0p, b1.reshape(1, 32), W2, b2.reshape(1, 64))
    return out.reshape(64)
